# baseline (speedup 1.0000x reference)
import functools

import jax
import jax.numpy as jnp
from jax import lax
from jax.experimental import pallas as pl
from jax.experimental.pallas import tpu as pltpu
from jax.experimental.pallas import tpu_sc as plsc

BATCH = 16384
D = 64
L = 16
IDX_CHUNK = 128
N_PASSES = 2
N_ROWS_HALF = 500000


def _build_sc_call():
    mesh = plsc.VectorSubcoreMesh(core_axis_name="c", subcore_axis_name="s")
    nc, ns = mesh.num_cores, mesh.num_subcores
    b_per_w = BATCH // (nc * ns)
    rows_per_pass = b_per_w // N_PASSES
    chunks_per_pass = rows_per_pass // IDX_CHUNK
    groups_per_pass = rows_per_pass // L

    @functools.partial(
        pl.kernel,
        out_type=jax.ShapeDtypeStruct((BATCH,), jnp.float32),
        mesh=mesh,
        scratch_types=[
            pltpu.VMEM((b_per_w,), jnp.int32),
            pltpu.VMEM((b_per_w,), jnp.int32),
            pltpu.VMEM((b_per_w,), jnp.int32),
            pltpu.VMEM((b_per_w,), jnp.int32),
            pltpu.VMEM((rows_per_pass, 2 * D), jnp.float32),
            pltpu.VMEM((rows_per_pass, 2 * D), jnp.float32),
            pltpu.VMEM((b_per_w,), jnp.float32),
            pltpu.VMEM((D, L), jnp.float32),
            pltpu.VMEM((L,), jnp.float32),
            pltpu.SemaphoreType.DMA,
        ],
        compiler_params=pltpu.CompilerParams(
            use_tc_tiling_on_sc=True, needs_layout_passes=False),
    )
    def neumf_kernel(uidx_hbm, iidx_hbm, uemb_hbm, iemb_hbm, w_hbm, b_hbm,
                     out_hbm, idx_u, idx_i, row_u, row_i, u_rows, i_rows,
                     out_v, w_v, b_v, sem):
        wid = lax.axis_index("s") * nc + lax.axis_index("c")
        base = wid * b_per_w
        pltpu.sync_copy(uidx_hbm.at[pl.ds(base, b_per_w)], idx_u)
        pltpu.sync_copy(iidx_hbm.at[pl.ds(base, b_per_w)], idx_i)
        pltpu.sync_copy(w_hbm, w_v)
        pltpu.sync_copy(b_hbm, b_v)

        def vrow_body(k, carry):
            sl = pl.ds(k * L, L)
            row_u[sl] = idx_u[sl] >> 1
            row_i[sl] = idx_i[sl] >> 1
            return carry

        lax.fori_loop(0, b_per_w // L, vrow_body, 0)

        lane = lax.iota(jnp.int32, L)
        bd = b_v[...]

        for p in range(N_PASSES):
            copies = []
            for c in range(chunks_per_pass):
                src = pl.ds(p * rows_per_pass + c * IDX_CHUNK, IDX_CHUNK)
                dst = pl.ds(c * IDX_CHUNK, IDX_CHUNK)
                copies.append(pltpu.async_copy(
                    uemb_hbm.at[row_u.at[src]], u_rows.at[dst], sem))
                copies.append(pltpu.async_copy(
                    iemb_hbm.at[row_i.at[src]], i_rows.at[dst], sem))
            for cp in copies:
                cp.wait()

            def group_body(g, carry):
                sl = pl.ds(p * rows_per_pass + g * L, L)
                off_u = (idx_u[sl] & 1) * D
                off_i = (idx_i[sl] & 1) * D
                rowv = g * L + lane
                acc = bd
                for f in range(D):
                    gu = plsc.load_gather(u_rows, [rowv, off_u + f])
                    gi = plsc.load_gather(i_rows, [rowv, off_i + f])
                    acc = acc + gu * gi * w_v[f]
                out_v[sl] = acc
                return carry

            lax.fori_loop(0, groups_per_pass, group_body, 0)

        pltpu.sync_copy(out_v, out_hbm.at[pl.ds(base, b_per_w)])

    return neumf_kernel


def kernel(users_index, items_index, user_mf_emb, item_mf_emb, W_pred, b_pred):
    u2 = user_mf_emb.reshape(N_ROWS_HALF, 2 * D)
    i2 = item_mf_emb.reshape(N_ROWS_HALF, 2 * D)
    w_lanes = jnp.broadcast_to(W_pred.reshape(D, 1), (D, L))
    b_lane = jnp.full((L,), b_pred[0], dtype=jnp.float32)
    call = _build_sc_call()
    out = call(users_index.astype(jnp.int32), items_index.astype(jnp.int32),
               u2, i2, w_lanes, b_lane)
    return out.reshape(BATCH, 1)


# speedup vs baseline: 2.0831x; 2.0831x over previous
import functools

import jax
import jax.numpy as jnp
from jax import lax
from jax.experimental import pallas as pl
from jax.experimental.pallas import tpu as pltpu
from jax.experimental.pallas import tpu_sc as plsc

BATCH = 16384
D = 64
L = 16
NROWS = 1000000
BLK = 128
LAST_TC = (NROWS // BLK) - 1
EDGE0 = (NROWS // BLK) * BLK


def _build_sc_call():
    mesh = plsc.VectorSubcoreMesh(core_axis_name="c", subcore_axis_name="s")
    nc, ns = mesh.num_cores, mesh.num_subcores
    b_per_w = BATCH // (nc * ns)
    n_pairs = b_per_w // 2

    @functools.partial(
        pl.kernel,
        out_type=jax.ShapeDtypeStruct((BATCH,), jnp.float32),
        mesh=mesh,
        scratch_types=[
            pltpu.VMEM((b_per_w + L,), jnp.int32),
            pltpu.VMEM((b_per_w + L,), jnp.int32),
            pltpu.VMEM((3, D, BLK), jnp.float32),
            pltpu.VMEM((3, D, BLK), jnp.float32),
            pltpu.VMEM((b_per_w,), jnp.float32),
            pltpu.VMEM((D,), jnp.float32),
            pltpu.VMEM((L,), jnp.float32),
            pltpu.SemaphoreType.DMA,
            pltpu.SemaphoreType.DMA,
        ],
        compiler_params=pltpu.CompilerParams(
            use_tc_tiling_on_sc=True, needs_layout_passes=False),
    )
    def neumf_kernel(uidx_hbm, iidx_hbm, ut_hbm, it_hbm, uedge_hbm, iedge_hbm,
                     w_hbm, b_hbm, out_hbm, idx_u, idx_i, u_all, i_all, out_v,
                     w_v, b_v, sem0, sem1):
        wid = lax.axis_index("s") * nc + lax.axis_index("c")
        base = wid * b_per_w
        pltpu.sync_copy(uidx_hbm.at[pl.ds(base, b_per_w)],
                        idx_u.at[pl.ds(0, b_per_w)])
        pltpu.sync_copy(iidx_hbm.at[pl.ds(base, b_per_w)],
                        idx_i.at[pl.ds(0, b_per_w)])

        def sidx(ref, e):
            return ref[pl.ds(e, L)][0]
        pltpu.sync_copy(w_hbm, w_v)
        pltpu.sync_copy(b_hbm, b_v)
        pltpu.sync_copy(uedge_hbm, u_all.at[2])
        pltpu.sync_copy(iedge_hbm, i_all.at[2])

        sems = (sem0, sem1)
        lane = lax.iota(jnp.int32, L)
        perms = [jnp.bitwise_xor(lane, d) for d in (1, 2, 4, 8)]
        dnums = lax.GatherDimensionNumbers(
            offset_dims=(), collapsed_slice_dims=(0,), start_index_map=(0,))

        def lane_sum(s):
            for p in perms:
                s = s + lax.gather(s, p[:, None], dnums, (1,),
                                   mode=lax.GatherScatterMode.PROMISE_IN_BOUNDS)
            return s

        w_chunks = [w_v[pl.ds(c * L, L)] for c in range(D // L)]
        bd = b_v[...]

        def fire(e, par):
            tcu = jnp.minimum(sidx(idx_u, e) >> 7, LAST_TC)
            tci = jnp.minimum(sidx(idx_i, e) >> 7, LAST_TC)
            pltpu.async_copy(ut_hbm.at[:, pl.ds(tcu * BLK, BLK)],
                             u_all.at[par], sems[par])
            pltpu.async_copy(it_hbm.at[:, pl.ds(tci * BLK, BLK)],
                             i_all.at[par], sems[par])

        def drain(par):
            dummy = ut_hbm.at[:, pl.ds(0, BLK)]
            pltpu.make_async_copy(dummy, u_all.at[par], sems[par]).wait()
            pltpu.make_async_copy(dummy, i_all.at[par], sems[par]).wait()

        def element_value(e, par):
            ru = sidx(idx_u, e)
            ri = sidx(idx_i, e)
            srcu = jnp.full((L,), jnp.where(ru >= EDGE0, 2, par), jnp.int32)
            srci = jnp.full((L,), jnp.where(ri >= EDGE0, 2, par), jnp.int32)
            rcu = jnp.full((L,), ru & (BLK - 1), jnp.int32)
            rci = jnp.full((L,), ri & (BLK - 1), jnp.int32)
            s = bd
            for c in range(D // L):
                fv = c * L + lane
                gu = plsc.load_gather(u_all, [srcu, fv, rcu])
                gi = plsc.load_gather(i_all, [srci, fv, rci])
                s = s + gu * gi * w_chunks[c]
            return lane_sum(s)

        fire(0, 0)
        fire(1, 1)

        def pair_body(t, acc):
            e0 = 2 * t
            drain(0)
            v0 = element_value(e0, 0)
            drain(1)
            v1 = element_value(e0 + 1, 1)
            nxt = jnp.minimum(e0 + 2, b_per_w - 2)
            fire(nxt, 0)
            fire(nxt + 1, 1)
            acc = jnp.where(lane == (e0 & 15), v0, acc)
            acc = jnp.where(lane == ((e0 + 1) & 15), v1, acc)

            @pl.when((t & 7) == 7)
            def _():
                out_v[pl.ds((t >> 3) * L, L)] = acc

            return jnp.where(jnp.full((L,), (t & 7) == 7), jnp.zeros_like(acc),
                             acc)

        lax.fori_loop(0, n_pairs, pair_body, jnp.zeros((L,), jnp.float32))
        drain(0)
        drain(1)

        pltpu.sync_copy(out_v, out_hbm.at[pl.ds(base, b_per_w)])

    return neumf_kernel


def kernel(users_index, items_index, user_mf_emb, item_mf_emb, W_pred, b_pred):
    ut = user_mf_emb.T
    it = item_mf_emb.T
    uedge = jnp.pad(user_mf_emb[EDGE0:].T, ((0, 0), (0, BLK - (NROWS - EDGE0))))
    iedge = jnp.pad(item_mf_emb[EDGE0:].T, ((0, 0), (0, BLK - (NROWS - EDGE0))))
    w_flat = W_pred.reshape(D)
    b_lane = jnp.full((L,), b_pred[0] / L, dtype=jnp.float32)
    call = _build_sc_call()
    out = call(users_index.astype(jnp.int32), items_index.astype(jnp.int32),
               ut, it, uedge, iedge, w_flat, b_lane)
    return out.reshape(BATCH, 1)


# speedup vs baseline: 2.6755x; 1.2844x over previous
import functools

import jax
import jax.numpy as jnp
from jax import lax
from jax.experimental import pallas as pl
from jax.experimental.pallas import tpu as pltpu
from jax.experimental.pallas import tpu_sc as plsc

BATCH = 16384
D = 64
L = 16
NROWS = 1000000
BLK = 128
LAST_TC = (NROWS // BLK) - 1
EDGE0 = (NROWS // BLK) * BLK


def _build_sc_call():
    mesh = plsc.VectorSubcoreMesh(core_axis_name="c", subcore_axis_name="s")
    nc, ns = mesh.num_cores, mesh.num_subcores
    b_per_w = BATCH // (nc * ns)
    n_pairs = b_per_w // 2

    @functools.partial(
        pl.kernel,
        out_type=jax.ShapeDtypeStruct((BATCH,), jnp.float32),
        mesh=mesh,
        scratch_types=[
            pltpu.VMEM((b_per_w + L,), jnp.int32),
            pltpu.VMEM((b_per_w + L,), jnp.int32),
            pltpu.VMEM((5, D, BLK), jnp.float32),
            pltpu.VMEM((5, D, BLK), jnp.float32),
            pltpu.VMEM((b_per_w,), jnp.float32),
            pltpu.VMEM((D,), jnp.float32),
            pltpu.VMEM((L,), jnp.float32),
            pltpu.SemaphoreType.DMA,
            pltpu.SemaphoreType.DMA,
            pltpu.SemaphoreType.DMA,
            pltpu.SemaphoreType.DMA,
        ],
        compiler_params=pltpu.CompilerParams(
            use_tc_tiling_on_sc=True, needs_layout_passes=False),
    )
    def neumf_kernel(uidx_hbm, iidx_hbm, ut_hbm, it_hbm, uedge_hbm, iedge_hbm,
                     w_hbm, b_hbm, out_hbm, idx_u, idx_i, u_all, i_all, out_v,
                     w_v, b_v, sem0, sem1, sem2, sem3):
        wid = lax.axis_index("s") * nc + lax.axis_index("c")
        base = wid * b_per_w
        pltpu.sync_copy(uidx_hbm.at[pl.ds(base, b_per_w)],
                        idx_u.at[pl.ds(0, b_per_w)])
        pltpu.sync_copy(iidx_hbm.at[pl.ds(base, b_per_w)],
                        idx_i.at[pl.ds(0, b_per_w)])

        def sidx(ref, e):
            return ref[pl.ds(e, L)][0]
        pltpu.sync_copy(w_hbm, w_v)
        pltpu.sync_copy(b_hbm, b_v)
        pltpu.sync_copy(uedge_hbm, u_all.at[4])
        pltpu.sync_copy(iedge_hbm, i_all.at[4])

        sems = (sem0, sem1, sem2, sem3)
        lane = lax.iota(jnp.int32, L)
        perms = [jnp.bitwise_xor(lane, d) for d in (1, 2, 4, 8)]
        dnums = lax.GatherDimensionNumbers(
            offset_dims=(), collapsed_slice_dims=(0,), start_index_map=(0,))

        def lane_sum(s):
            for p in perms:
                s = s + lax.gather(s, p[:, None], dnums, (1,),
                                   mode=lax.GatherScatterMode.PROMISE_IN_BOUNDS)
            return s

        w_chunks = [w_v[pl.ds(c * L, L)] for c in range(D // L)]
        bd = b_v[...]

        def fire(e, par):
            tcu = jnp.minimum(sidx(idx_u, e) >> 7, LAST_TC)
            tci = jnp.minimum(sidx(idx_i, e) >> 7, LAST_TC)
            pltpu.async_copy(ut_hbm.at[:, pl.ds(tcu * BLK, BLK)],
                             u_all.at[par], sems[par])
            pltpu.async_copy(it_hbm.at[:, pl.ds(tci * BLK, BLK)],
                             i_all.at[par], sems[par])

        def drain(par):
            dummy = ut_hbm.at[:, pl.ds(0, BLK)]
            pltpu.make_async_copy(dummy, u_all.at[par], sems[par]).wait()
            pltpu.make_async_copy(dummy, i_all.at[par], sems[par]).wait()

        def element_value(e, par):
            ru = sidx(idx_u, e)
            ri = sidx(idx_i, e)
            srcu = jnp.full((L,), jnp.where(ru >= EDGE0, 4, par), jnp.int32)
            srci = jnp.full((L,), jnp.where(ri >= EDGE0, 4, par), jnp.int32)
            rcu = jnp.full((L,), ru & (BLK - 1), jnp.int32)
            rci = jnp.full((L,), ri & (BLK - 1), jnp.int32)
            s = bd
            for c in range(D // L):
                fv = c * L + lane
                gu = plsc.load_gather(u_all, [srcu, fv, rcu])
                gi = plsc.load_gather(i_all, [srci, fv, rci])
                s = s + gu * gi * w_chunks[c]
            return lane_sum(s)

        NBUF = 4
        for par in range(NBUF):
            fire(par, par)

        def quad_body(t, acc):
            e0 = NBUF * t
            for par in range(NBUF):
                e = e0 + par
                drain(par)
                v = element_value(e, par)
                fire(jnp.minimum(e + NBUF, b_per_w - 1), par)
                acc = jnp.where(lane == (e & 15), v, acc)

            @pl.when((t & 3) == 3)
            def _():
                out_v[pl.ds((t >> 2) * L, L)] = acc

            return jnp.where(jnp.full((L,), (t & 3) == 3), jnp.zeros_like(acc),
                             acc)

        lax.fori_loop(0, b_per_w // NBUF, quad_body,
                      jnp.zeros((L,), jnp.float32))
        for par in range(NBUF):
            drain(par)

        pltpu.sync_copy(out_v, out_hbm.at[pl.ds(base, b_per_w)])

    return neumf_kernel


def kernel(users_index, items_index, user_mf_emb, item_mf_emb, W_pred, b_pred):
    ut = user_mf_emb.T
    it = item_mf_emb.T
    uedge = jnp.pad(user_mf_emb[EDGE0:].T, ((0, 0), (0, BLK - (NROWS - EDGE0))))
    iedge = jnp.pad(item_mf_emb[EDGE0:].T, ((0, 0), (0, BLK - (NROWS - EDGE0))))
    w_flat = W_pred.reshape(D)
    b_lane = jnp.full((L,), b_pred[0] / L, dtype=jnp.float32)
    call = _build_sc_call()
    out = call(users_index.astype(jnp.int32), items_index.astype(jnp.int32),
               ut, it, uedge, iedge, w_flat, b_lane)
    return out.reshape(BATCH, 1)
